# baseline (speedup 1.0000x reference)
import jax
import jax.numpy as jnp
from jax import lax
from jax.experimental import pallas as pl
from jax.experimental.pallas import tpu as pltpu
from jax.experimental.pallas import tpu_sc as plsc

_BB = 128
_NB = 4


def kernel(input_ids, weight):
    batch, seq = input_ids.shape
    emb_dim = weight.shape[1]
    idx_t = input_ids.T.astype(jnp.int32)

    info = plsc.get_sparse_core_info()
    nw = info.num_cores * info.num_subcores
    assert batch // nw == _BB and seq % _NB == 0

    mesh = plsc.VectorSubcoreMesh(
        core_axis_name="core", subcore_axis_name="subcore"
    )
    n_cg = emb_dim // 8

    @pl.kernel(
        out_type=jax.ShapeDtypeStruct(
            (seq, n_cg, nw, 8, _BB), weight.dtype
        ),
        mesh=mesh,
        scratch_types=[
            pltpu.VMEM((seq, _BB), jnp.int32),
            pltpu.VMEM((_NB, _BB, emb_dim), weight.dtype),
            pltpu.VMEM((_NB, emb_dim, _BB + 1), weight.dtype),
        ]
        + [pltpu.SemaphoreType.DMA] * (2 * _NB),
        compiler_params=pltpu.CompilerParams(
            use_tc_tiling_on_sc=False, needs_layout_passes=False
        ),
    )
    def gather_kernel(table_hbm, idx_hbm, out_hbm, idx_v, rows, rows_t, *sems):
        sgs, sos = sems[:_NB], sems[_NB:]
        wid = lax.axis_index("subcore") * info.num_cores + lax.axis_index(
            "core"
        )
        b0 = wid * _BB

        pltpu.sync_copy(idx_hbm.at[:, pl.ds(b0, _BB)], idx_v)

        def gather(s, b):
            return pltpu.async_copy(
                table_hbm.at[idx_v.at[s]], rows.at[b], sgs[b]
            )

        def out_descs(s, b):
            return [
                pltpu.make_async_copy(
                    rows_t.at[b, pl.ds(cg * 8, 8), pl.ds(0, _BB)],
                    out_hbm.at[s, cg, wid],
                    sos[b],
                )
                for cg in range(n_cg)
            ]

        for b in range(_NB - 1):
            gather(b, b)

        @pl.loop(0, seq // _NB)
        def _(p):
            s0 = _NB * p
            for b in range(_NB):
                s = s0 + b

                @pl.when(p > 0)
                def _():
                    for d in out_descs(s - _NB, b):
                        d.wait()

                pltpu.make_async_copy(
                    table_hbm.at[idx_v.at[s]], rows.at[b], sgs[b]
                ).wait()

                @pl.when(s + _NB - 1 < seq)
                def _():
                    gather(s + _NB - 1, (b + _NB - 1) % _NB)

                iota16 = lax.iota(jnp.int32, 16)
                half = 16
                for j0 in range(0, _BB, 8):
                    vals = [
                        (
                            rows.at[b].at[j][pl.ds(0, half)],
                            rows.at[b].at[j][pl.ds(half, half)],
                        )
                        for j in range(j0, j0 + 8)
                    ]
                    for j, (v0, v1) in zip(range(j0, j0 + 8), vals):
                        jvec = jnp.full((16,), j, dtype=jnp.int32)
                        plsc.store_scatter(rows_t.at[b], [iota16, jvec], v0)
                        plsc.store_scatter(
                            rows_t.at[b], [iota16 + half, jvec], v1
                        )

                for d in out_descs(s, b):
                    d.start()

        for b in range(_NB):
            for d in out_descs(seq - _NB + b, b):
                d.wait()

    out5 = gather_kernel(weight, idx_t)
    return out5.transpose(2, 4, 0, 1, 3).reshape(batch, seq, emb_dim)


# speedup vs baseline: 1.0691x; 1.0691x over previous
import jax
import jax.numpy as jnp
from jax import lax
from jax.experimental import pallas as pl
from jax.experimental.pallas import tpu as pltpu
from jax.experimental.pallas import tpu_sc as plsc

_BB = 128


def kernel(input_ids, weight):
    batch, seq = input_ids.shape
    emb_dim = weight.shape[1]
    idx_t = input_ids.T.astype(jnp.int32)

    info = plsc.get_sparse_core_info()
    nw = info.num_cores * info.num_subcores
    assert batch // nw == _BB

    mesh = plsc.VectorSubcoreMesh(
        core_axis_name="core", subcore_axis_name="subcore"
    )

    n_cg = emb_dim // 8

    @pl.kernel(
        out_type=jax.ShapeDtypeStruct(
            (seq, n_cg, nw, 8, _BB), weight.dtype
        ),
        mesh=mesh,
        scratch_types=[
            pltpu.VMEM((seq, _BB), jnp.int32),
            pltpu.VMEM((2, _BB, emb_dim), weight.dtype),
            pltpu.VMEM((2, emb_dim, _BB + 1), weight.dtype),
            pltpu.SemaphoreType.DMA,
            pltpu.SemaphoreType.DMA,
            pltpu.SemaphoreType.DMA,
            pltpu.SemaphoreType.DMA,
        ],
        compiler_params=pltpu.CompilerParams(
            use_tc_tiling_on_sc=False, needs_layout_passes=False
        ),
    )
    def gather_kernel(
        table_hbm, idx_hbm, out_hbm, idx_v, rows, rows_t, sg0, sg1, so0, so1
    ):
        wid = lax.axis_index("subcore") * info.num_cores + lax.axis_index(
            "core"
        )
        b0 = wid * _BB
        sgs = (sg0, sg1)
        sos = (so0, so1)

        pltpu.sync_copy(idx_hbm.at[:, pl.ds(b0, _BB)], idx_v)

        def gather(s, b):
            return pltpu.async_copy(
                table_hbm.at[idx_v.at[s]], rows.at[b], sgs[b]
            )

        def out_descs(s, b):
            return [
                pltpu.make_async_copy(
                    rows_t.at[b, pl.ds(cg * 8, 8), pl.ds(0, _BB)],
                    out_hbm.at[s, cg, wid],
                    sos[b],
                )
                for cg in range(n_cg)
            ]

        gather(0, 0)
        gather(1, 1)

        @pl.loop(0, seq // 2)
        def _(p):
            s0 = 2 * p
            for b in range(2):
                s = s0 + b

                @pl.when(p > 0)
                def _():
                    for d in out_descs(s - 2, b):
                        d.wait()

                pltpu.make_async_copy(
                    table_hbm.at[idx_v.at[s]], rows.at[b], sgs[b]
                ).wait()
                iota16 = lax.iota(jnp.int32, 16)
                half = emb_dim // 2
                for j0 in range(0, _BB, 8):
                    vals = [
                        (
                            rows.at[b].at[j][pl.ds(0, half)],
                            rows.at[b].at[j][pl.ds(half, half)],
                        )
                        for j in range(j0, j0 + 8)
                    ]
                    for j, (v0, v1) in zip(range(j0, j0 + 8), vals):
                        jvec = jnp.full((16,), j, dtype=jnp.int32)
                        plsc.store_scatter(rows_t.at[b], [iota16, jvec], v0)
                        plsc.store_scatter(
                            rows_t.at[b], [iota16 + half, jvec], v1
                        )

                @pl.when(p < seq // 2 - 1)
                def _():
                    gather(s + 2, b)

                for d in out_descs(s, b):
                    d.start()

        for d in out_descs(seq - 2, 0):
            d.wait()
        for d in out_descs(seq - 1, 1):
            d.wait()

    out5 = gather_kernel(weight, idx_t)
    return out5.transpose(2, 4, 0, 1, 3).reshape(batch, seq, emb_dim)
